# baseline (speedup 1.0000x reference)
import jax
import jax.numpy as jnp
from jax import lax
from jax.experimental import pallas as pl
from jax.experimental.pallas import tpu as pltpu
from jax.experimental.pallas import tpu_sc as plsc

B, S, D = 1024, 200, 128
L = 16
NC, NS = 2, 16
NW = NC * NS
RW = B // NW
NG = S // L
TAIL = S - NG * L
NCH = D // L
CAPR = 208
CAP = CAPR + 2 * L
SIZES = (128, 64, 32, 16)


def _body(ids_hbm, mask_hbm, w_hbm, out_hbm,
          ids_all, mask_all, cids_a, cids_b, cids_c, cids_d,
          buf_a, buf_b, buf_c, buf_d, out_all,
          sem_a, sem_b, sem_c, sem_d):
    wid = lax.axis_index("s") * NC + lax.axis_index("c")
    base = wid * RW
    pltpu.sync_copy(ids_hbm.at[pl.ds(base * S, RW * S)], ids_all)
    pltpu.sync_copy(mask_hbm.at[pl.ds(base * S, RW * S)], mask_all)

    iota = lax.iota(jnp.int32, L)
    shuf = [jnp.maximum(iota - s, 0) for s in (1, 2, 4, 8)]

    def prefix_incl(x):
        for s, idx in zip((1, 2, 4, 8), shuf):
            sh = x.at[idx].get(mode="promise_in_bounds")
            x = x + jnp.where(iota >= s, sh, 0)
        return x

    pad_ids = wid * L + iota
    lane15 = jnp.full((L,), L - 1, jnp.int32)

    def compress(r, cids):
        off = r * S
        vis, idss = [], []
        for g in range(NG):
            o = off + g * L
            vis.append(mask_all[pl.ds(o, L)])
            idss.append(ids_all[pl.ds(o, L)])
        o = off + S - L
        vis.append(jnp.where(iota >= L - TAIL, mask_all[pl.ds(o, L)], 0))
        idss.append(ids_all[pl.ds(o, L)])

        incls = list(vis)
        for s, idx in zip((1, 2, 4, 8), shuf):
            shifted = [x.at[idx].get(mode="promise_in_bounds") for x in incls]
            incls = [x + jnp.where(iota >= s, sh, 0)
                     for x, sh in zip(incls, shifted)]
        tots = [x.at[lane15].get(mode="promise_in_bounds") for x in incls]
        cvec = jnp.zeros((L,), jnp.int32)
        bases = []
        for g in range(NG + 1):
            bases.append(cvec)
            cvec = cvec + tots[g]
        for g in range(NG + 1):
            dest = bases[g] + (incls[g] - vis[g])
            plsc.store_scatter(cids, [dest], idss[g], mask=vis[g] != 0)
        plsc.store_scatter(cids, [cvec + iota], pad_ids)
        return cvec[0]

    def streams(cids, buf, c, sem):
        n = lax.shift_right_logical(c + 15, 4)
        res = []
        o = jnp.int32(0)
        for sz in SIZES:
            bit = sz // L
            on = lax.bitwise_and(n, bit) != 0
            cp = pltpu.make_async_copy(w_hbm.at[cids.at[pl.ds(o, sz)]],
                                       buf.at[pl.ds(o, sz)], sem)
            res.append((on, cp))
            o = o + jnp.where(on, sz, 0)
        return res

    def fire(cids, buf, c, sem):
        for on, cp in streams(cids, buf, c, sem):
            @pl.when(on)
            def _():
                cp.start()

    def accumulate(r, buf, cids, c, sem):
        for on, cp in streams(cids, buf, c, sem):
            @pl.when(on)
            def _():
                cp.wait()

        n_full = lax.shift_right_logical(c, 4)

        def acc_body(g, a):
            t0 = g * L
            for k in range(L):
                a = tuple(a[ch] + buf[t0 + k, pl.ds(ch * L, L)]
                          for ch in range(NCH))
            return a
        acc = lax.fori_loop(0, n_full, acc_body,
                            (jnp.zeros((L,), jnp.float32),) * NCH)

        tl = lax.bitwise_and(c, 15)
        t0 = c - tl
        tlv = jnp.full((L,), tl, jnp.int32)
        zf = jnp.zeros((L,), jnp.float32)
        for k in range(L):
            keep = tlv > k
            acc = tuple(acc[ch] + jnp.where(keep, buf[t0 + k, pl.ds(ch * L, L)], zf)
                        for ch in range(NCH))

        inv = jnp.ones((L,), jnp.float32) / jnp.maximum(
            jnp.full((L,), c, jnp.int32).astype(jnp.float32), 1.0)
        for ch in range(NCH):
            out_all[pl.ds(r * D + ch * L, L)] = acc[ch] * inv

    c0 = compress(0, cids_a)
    fire(cids_a, buf_a, c0, sem_a)
    c1 = compress(1, cids_b)
    fire(cids_b, buf_b, c1, sem_b)

    def iter_body(i, carry):
        c_a, c_b = carry
        r0 = 4 * i
        c_c = compress(r0 + 2, cids_c)
        fire(cids_c, buf_c, c_c, sem_c)
        accumulate(r0, buf_a, cids_a, c_a, sem_a)
        c_d = compress(r0 + 3, cids_d)
        fire(cids_d, buf_d, c_d, sem_d)
        accumulate(r0 + 1, buf_b, cids_b, c_b, sem_b)
        c_a2 = compress(jnp.minimum(r0 + 4, RW - 1), cids_a)

        @pl.when(i < RW // 4 - 1)
        def _():
            fire(cids_a, buf_a, c_a2, sem_a)
        accumulate(r0 + 2, buf_c, cids_c, c_c, sem_c)
        c_b2 = compress(jnp.minimum(r0 + 5, RW - 1), cids_b)

        @pl.when(i < RW // 4 - 1)
        def _():
            fire(cids_b, buf_b, c_b2, sem_b)
        accumulate(r0 + 3, buf_d, cids_d, c_d, sem_d)
        return (c_a2, c_b2)

    lax.fori_loop(0, RW // 4, iter_body, (c0, c1))
    pltpu.sync_copy(out_all, out_hbm.at[pl.ds(base * D, RW * D)])


def kernel(input_ids, attention_mask, W):
    mesh = plsc.VectorSubcoreMesh(core_axis_name="c", subcore_axis_name="s")
    k = pl.kernel(
        _body,
        out_type=jax.ShapeDtypeStruct((B * D,), jnp.float32),
        mesh=mesh,
        compiler_params=pltpu.CompilerParams(needs_layout_passes=False),
        scratch_types=[
            pltpu.VMEM((RW * S,), jnp.int32),
            pltpu.VMEM((RW * S,), jnp.int32),
            pltpu.VMEM((CAP,), jnp.int32),
            pltpu.VMEM((CAP,), jnp.int32),
            pltpu.VMEM((CAP,), jnp.int32),
            pltpu.VMEM((CAP,), jnp.int32),
            pltpu.VMEM((CAPR, D), jnp.float32),
            pltpu.VMEM((CAPR, D), jnp.float32),
            pltpu.VMEM((CAPR, D), jnp.float32),
            pltpu.VMEM((CAPR, D), jnp.float32),
            pltpu.VMEM((RW * D,), jnp.float32),
            pltpu.SemaphoreType.DMA,
            pltpu.SemaphoreType.DMA,
            pltpu.SemaphoreType.DMA,
            pltpu.SemaphoreType.DMA,
        ],
    )
    out = k(input_ids.astype(jnp.int32).reshape(-1),
            attention_mask.astype(jnp.int32).reshape(-1), W)
    return out.reshape(B, D)


# speedup vs baseline: 1.1366x; 1.1366x over previous
import jax
import jax.numpy as jnp
from jax import lax
from jax.experimental import pallas as pl
from jax.experimental.pallas import tpu as pltpu
from jax.experimental.pallas import tpu_sc as plsc

B, S, D = 1024, 200, 128
L = 16
NC, NS = 2, 16
NW = NC * NS
RW = B // NW
NG = S // L
TAIL = S - NG * L
NCH = D // L
CAPR = 208
CAP = CAPR + 2 * L
SIZES = (128, 64, 32, 16)


def _body(ids_hbm, mask_hbm, w_hbm, out_hbm,
          ids_all, mask_all, cids_a, cids_b, buf_a, buf_b, out_all,
          sem_a, sem_b):
    wid = lax.axis_index("s") * NC + lax.axis_index("c")
    base = wid * RW
    pltpu.sync_copy(ids_hbm.at[pl.ds(base * S, RW * S)], ids_all)
    pltpu.sync_copy(mask_hbm.at[pl.ds(base * S, RW * S)], mask_all)

    iota = lax.iota(jnp.int32, L)
    shuf = [jnp.maximum(iota - s, 0) for s in (1, 2, 4, 8)]

    def prefix_incl(x):
        for s, idx in zip((1, 2, 4, 8), shuf):
            sh = x.at[idx].get(mode="promise_in_bounds")
            x = x + jnp.where(iota >= s, sh, 0)
        return x

    pad_ids = wid * L + iota
    lane15 = jnp.full((L,), L - 1, jnp.int32)

    def compress(r, cids):
        off = r * S
        vis, idss = [], []
        for g in range(NG):
            o = off + g * L
            vis.append(mask_all[pl.ds(o, L)])
            idss.append(ids_all[pl.ds(o, L)])
        o = off + S - L
        vis.append(jnp.where(iota >= L - TAIL, mask_all[pl.ds(o, L)], 0))
        idss.append(ids_all[pl.ds(o, L)])

        incls = list(vis)
        for s, idx in zip((1, 2, 4, 8), shuf):
            shifted = [x.at[idx].get(mode="promise_in_bounds") for x in incls]
            incls = [x + jnp.where(iota >= s, sh, 0)
                     for x, sh in zip(incls, shifted)]
        tots = [x.at[lane15].get(mode="promise_in_bounds") for x in incls]
        cvec = jnp.zeros((L,), jnp.int32)
        bases = []
        for g in range(NG + 1):
            bases.append(cvec)
            cvec = cvec + tots[g]
        for g in range(NG + 1):
            dest = bases[g] + (incls[g] - vis[g])
            plsc.store_scatter(cids, [dest], idss[g], mask=vis[g] != 0)
        plsc.store_scatter(cids, [cvec + iota], pad_ids)
        return cvec[0]

    def streams(cids, buf, c, sem):
        n = lax.shift_right_logical(c + 15, 4)
        res = []
        o = jnp.int32(0)
        for sz in SIZES:
            bit = sz // L
            on = lax.bitwise_and(n, bit) != 0
            cp = pltpu.make_async_copy(w_hbm.at[cids.at[pl.ds(o, sz)]],
                                       buf.at[pl.ds(o, sz)], sem)
            res.append((on, cp))
            o = o + jnp.where(on, sz, 0)
        return res

    def fire(cids, buf, c, sem):
        for on, cp in streams(cids, buf, c, sem):
            @pl.when(on)
            def _():
                cp.start()

    def accumulate(r, buf, cids, c, sem):
        for on, cp in streams(cids, buf, c, sem):
            @pl.when(on)
            def _():
                cp.wait()

        n_full = lax.shift_right_logical(c, 4)

        def acc_body(g, a):
            t0 = g * L
            for k in range(L):
                a = tuple(a[ch] + buf[t0 + k, pl.ds(ch * L, L)]
                          for ch in range(NCH))
            return a
        acc = lax.fori_loop(0, n_full, acc_body,
                            (jnp.zeros((L,), jnp.float32),) * NCH)

        tl = lax.bitwise_and(c, 15)
        t0 = c - tl
        tlv = jnp.full((L,), tl, jnp.int32)
        zf = jnp.zeros((L,), jnp.float32)
        for k in range(L):
            keep = tlv > k
            acc = tuple(acc[ch] + jnp.where(keep, buf[t0 + k, pl.ds(ch * L, L)], zf)
                        for ch in range(NCH))

        inv = jnp.ones((L,), jnp.float32) / jnp.maximum(
            jnp.full((L,), c, jnp.int32).astype(jnp.float32), 1.0)
        for ch in range(NCH):
            out_all[pl.ds(r * D + ch * L, L)] = acc[ch] * inv

    c0 = compress(0, cids_a)
    fire(cids_a, buf_a, c0, sem_a)

    def iter_body(i, c_a):
        r0 = 2 * i
        c_b = compress(r0 + 1, cids_b)
        fire(cids_b, buf_b, c_b, sem_b)
        accumulate(r0, buf_a, cids_a, c_a, sem_a)
        rr = jnp.minimum(r0 + 2, RW - 1)
        c_a2 = compress(rr, cids_a)

        @pl.when(i < RW // 2 - 1)
        def _():
            fire(cids_a, buf_a, c_a2, sem_a)
        accumulate(r0 + 1, buf_b, cids_b, c_b, sem_b)
        return c_a2

    lax.fori_loop(0, RW // 2, iter_body, c0)
    pltpu.sync_copy(out_all, out_hbm.at[pl.ds(base * D, RW * D)])


def kernel(input_ids, attention_mask, W):
    mesh = plsc.VectorSubcoreMesh(core_axis_name="c", subcore_axis_name="s")
    k = pl.kernel(
        _body,
        out_type=jax.ShapeDtypeStruct((B * D,), jnp.float32),
        mesh=mesh,
        compiler_params=pltpu.CompilerParams(needs_layout_passes=False),
        scratch_types=[
            pltpu.VMEM((RW * S,), jnp.int32),
            pltpu.VMEM((RW * S,), jnp.int32),
            pltpu.VMEM((CAP,), jnp.int32),
            pltpu.VMEM((CAP,), jnp.int32),
            pltpu.VMEM((CAPR, D), jnp.float32),
            pltpu.VMEM((CAPR, D), jnp.float32),
            pltpu.VMEM((RW * D,), jnp.float32),
            pltpu.SemaphoreType.DMA,
            pltpu.SemaphoreType.DMA,
        ],
    )
    out = k(input_ids.astype(jnp.int32).reshape(-1),
            attention_mask.astype(jnp.int32).reshape(-1), W)
    return out.reshape(B, D)
